# baseline (speedup 1.0000x reference)
import functools

import jax
import jax.numpy as jnp
from jax import lax
from jax.experimental import pallas as pl
from jax.experimental.pallas import tpu as pltpu
from jax.experimental.pallas import tpu_sc as plsc

_VOCAB = 1000000
_EMBED_DIM = 32
_BATCH = 4096
_N_FIELDS = 26
_WIDE_IN = 1024
_WIDE_OUT = 64
_H1 = 256
_H2 = 128
_DEEP_OUT = 64

_B_FLAT = _BATCH * _N_FIELDS

_NC = 2
_NS = 16
_NW = _NC * _NS
_B_PER_W = _B_FLAT // _NW


def _sc_gather_t(table_t, idx):
    mesh = plsc.VectorSubcoreMesh(core_axis_name="c", subcore_axis_name="s")

    @functools.partial(
        pl.kernel,
        mesh=mesh,
        out_type=jax.ShapeDtypeStruct((_EMBED_DIM, _B_FLAT), jnp.float32),
        compiler_params=pltpu.CompilerParams(use_tc_tiling_on_sc=False),
        scratch_types=[
            pltpu.VMEM((_B_PER_W,), jnp.int32),
            pltpu.VMEM((_EMBED_DIM, _B_PER_W), jnp.float32),
            pltpu.SemaphoreType.DMA,
        ],
    )
    def k(table_hbm, idx_hbm, out_hbm, idx_v, rows_v, sem):
        wid = lax.axis_index("s") * _NC + lax.axis_index("c")
        base = wid * _B_PER_W
        pltpu.sync_copy(idx_hbm.at[pl.ds(base, _B_PER_W)], idx_v)
        copies = [
            pltpu.async_copy(table_hbm.at[e].at[idx_v], rows_v.at[e], sem)
            for e in range(_EMBED_DIM)
        ]
        for c in copies:
            c.wait()
        pltpu.sync_copy(rows_v, out_hbm.at[:, pl.ds(base, _B_PER_W)])

    return k(table_t, idx)


_WIDE_BLK = 512


def _wide_body(x_ref, w_ref, b_ref, o_ref):
    y = (
        jnp.dot(x_ref[...], w_ref[...], preferred_element_type=jnp.float32)
        + b_ref[...]
    )
    o_ref[...] = y.T


def _tc_wide(wide_input, W_wide, b_wide):
    grid = (_BATCH // _WIDE_BLK,)
    return pl.pallas_call(
        _wide_body,
        grid=grid,
        in_specs=[
            pl.BlockSpec((_WIDE_BLK, _WIDE_IN), lambda i: (i, 0)),
            pl.BlockSpec((_WIDE_IN, _WIDE_OUT), lambda i: (0, 0)),
            pl.BlockSpec((1, _WIDE_OUT), lambda i: (0, 0)),
        ],
        out_specs=pl.BlockSpec((_WIDE_OUT, _WIDE_BLK), lambda i: (0, i)),
        out_shape=jax.ShapeDtypeStruct((_WIDE_OUT, _BATCH), jnp.float32),
    )(wide_input, W_wide, b_wide.reshape(1, _WIDE_OUT))


_MLP_BLK = 2048
_MLP_J = _BATCH // _MLP_BLK

_CT = (((0,), (0,)), ((), ()))


def _mlp_body(x_ref, w1_ref, b1_ref, w2_ref, b2_ref, w3_ref, b3_ref, o_ref):
    x = x_ref[...].astype(jnp.bfloat16)
    w1 = w1_ref[...].astype(jnp.bfloat16)
    w2 = w2_ref[...].astype(jnp.bfloat16)
    w3 = w3_ref[...].astype(jnp.bfloat16)
    h = jnp.maximum(
        lax.dot_general(w1, x, _CT, preferred_element_type=jnp.float32)
        + b1_ref[...],
        0.0,
    ).astype(jnp.bfloat16)
    h = jnp.maximum(
        lax.dot_general(w2, h, _CT, preferred_element_type=jnp.float32)
        + b2_ref[...],
        0.0,
    ).astype(jnp.bfloat16)
    y = lax.dot_general(w3, h, _CT, preferred_element_type=jnp.float32) + b3_ref[...]
    o_ref[...] = y[None]


def _tc_mlp(emb_t, W1, b1, W2, b2, W3, b3):
    grid = (_N_FIELDS, _MLP_J)
    return pl.pallas_call(
        _mlp_body,
        grid=grid,
        in_specs=[
            pl.BlockSpec(
                (_EMBED_DIM, _MLP_BLK),
                lambda f, j: (0, f * _MLP_J + j),
            ),
            pl.BlockSpec((_EMBED_DIM, _H1), lambda f, j: (0, 0)),
            pl.BlockSpec((_H1, 1), lambda f, j: (0, 0)),
            pl.BlockSpec((_H1, _H2), lambda f, j: (0, 0)),
            pl.BlockSpec((_H2, 1), lambda f, j: (0, 0)),
            pl.BlockSpec((_H2, _DEEP_OUT), lambda f, j: (0, 0)),
            pl.BlockSpec((_DEEP_OUT, 1), lambda f, j: (0, 0)),
        ],
        out_specs=pl.BlockSpec((1, _DEEP_OUT, _MLP_BLK), lambda f, j: (f, 0, j)),
        out_shape=jax.ShapeDtypeStruct((_N_FIELDS, _DEEP_OUT, _BATCH), jnp.float32),
    )(
        emb_t,
        W1,
        b1.reshape(_H1, 1),
        W2,
        b2.reshape(_H2, 1),
        W3,
        b3.reshape(_DEEP_OUT, 1),
    )


def kernel(wide_input, deep_input, table, W_wide, b_wide, W1, b1, W2, b2, W3, b3):
    idx = deep_input.astype(jnp.int32).T.reshape(_B_FLAT)
    emb_t = _sc_gather_t(table.T, idx)
    wide_t = _tc_wide(wide_input, W_wide, b_wide)
    deep_t = _tc_mlp(emb_t, W1, b1, W2, b2, W3, b3)
    wide_out = wide_t.T
    deep_out = jnp.transpose(deep_t, (2, 0, 1))
    return (wide_out, deep_out)


# speedup vs baseline: 7.2380x; 7.2380x over previous
import functools

import jax
import jax.numpy as jnp
from jax import lax
from jax.experimental import pallas as pl
from jax.experimental.pallas import tpu as pltpu
from jax.experimental.pallas import tpu_sc as plsc

_VOCAB = 1000000
_EMBED_DIM = 32
_BATCH = 4096
_N_FIELDS = 26
_WIDE_IN = 1024
_WIDE_OUT = 64
_H1 = 256
_H2 = 128
_DEEP_OUT = 64

_B_FLAT = _BATCH * _N_FIELDS
_B_PACK = _B_FLAT // 4

_NC = 2
_NS = 16
_NW = _NC * _NS
_B_PER_W = _B_FLAT // _NW


_TR_BLKV = 16384
_TR_STEPS = _VOCAB // _TR_BLKV
_TR_TAIL = _VOCAB - _TR_STEPS * _TR_BLKV


def _pack_body(x_ref, o_ref):
    eye_pad = jnp.concatenate(
        [
            jnp.eye(_EMBED_DIM, dtype=jnp.float32),
            jnp.zeros((_EMBED_DIM, 128 - _EMBED_DIM), dtype=jnp.float32),
        ],
        axis=1,
    )
    o_ref[...] = lax.dot_general(
        x_ref[...], eye_pad, (((0,), (0,)), ((), ())),
        preferred_element_type=jnp.float32,
    )


def _tc_table_pack(table_t):
    grid = (pl.cdiv(_VOCAB, _TR_BLKV),)
    return pl.pallas_call(
        _pack_body,
        grid=grid,
        in_specs=[pl.BlockSpec((_EMBED_DIM, _TR_BLKV), lambda i: (0, i))],
        out_specs=pl.BlockSpec((_TR_BLKV, 128), lambda i: (i, 0)),
        out_shape=jax.ShapeDtypeStruct((_VOCAB, 128), jnp.float32),
    )(table_t)


_G_CHUNKS = 4
_G_ROWS = _B_PER_W // _G_CHUNKS


def _sc_gather(table_rm, idx):
    mesh = plsc.VectorSubcoreMesh(core_axis_name="c", subcore_axis_name="s")

    @functools.partial(
        pl.kernel,
        mesh=mesh,
        out_type=jax.ShapeDtypeStruct((_B_FLAT, _EMBED_DIM), jnp.float32),
        compiler_params=pltpu.CompilerParams(use_tc_tiling_on_sc=False),
        scratch_types=[
            pltpu.VMEM((_B_PER_W,), jnp.int32),
            pltpu.VMEM((_G_ROWS, 128), jnp.float32),
            pltpu.SemaphoreType.DMA,
        ],
    )
    def k(table_hbm, idx_hbm, out_hbm, idx_v, rows_v, sem):
        wid = lax.axis_index("s") * _NC + lax.axis_index("c")
        base = wid * _B_PER_W
        pltpu.sync_copy(idx_hbm.at[pl.ds(base, _B_PER_W)], idx_v)
        for c in range(_G_CHUNKS):
            pltpu.async_copy(
                table_hbm.at[idx_v.at[pl.ds(c * _G_ROWS, _G_ROWS)]],
                rows_v,
                sem,
            ).wait()
            pltpu.sync_copy(
                rows_v.at[:, pl.ds(0, _EMBED_DIM)],
                out_hbm.at[pl.ds(base + c * _G_ROWS, _G_ROWS)],
            )

    return k(table_rm, idx)


_WIDE_BLK = 512


def _wide_body(x_ref, w_ref, b_ref, o_ref):
    y = (
        jnp.dot(x_ref[...], w_ref[...], preferred_element_type=jnp.float32)
        + b_ref[...]
    )
    o_ref[...] = y.T


def _tc_wide(wide_input, W_wide, b_wide):
    grid = (_BATCH // _WIDE_BLK,)
    return pl.pallas_call(
        _wide_body,
        grid=grid,
        in_specs=[
            pl.BlockSpec((_WIDE_BLK, _WIDE_IN), lambda i: (i, 0)),
            pl.BlockSpec((_WIDE_IN, _WIDE_OUT), lambda i: (0, 0)),
            pl.BlockSpec((1, _WIDE_OUT), lambda i: (0, 0)),
        ],
        out_specs=pl.BlockSpec((_WIDE_OUT, _WIDE_BLK), lambda i: (0, i)),
        out_shape=jax.ShapeDtypeStruct((_WIDE_OUT, _BATCH), jnp.float32),
    )(wide_input, W_wide, b_wide.reshape(1, _WIDE_OUT))


_MLP_BLK = 2048
_MLP_PBLK = _MLP_BLK // 4
_MLP_J = _BATCH // _MLP_BLK


def _mlp_body(x_ref, w1_ref, b1_ref, w2_ref, b2_ref, w3_ref, b3_ref, o_ref):
    w1 = w1_ref[...].astype(jnp.bfloat16)
    w2 = w2_ref[...].astype(jnp.bfloat16)
    w3 = w3_ref[...].astype(jnp.bfloat16)
    ys = []
    for k in range(4):
        x = x_ref[:, k * _EMBED_DIM : (k + 1) * _EMBED_DIM].astype(jnp.bfloat16)
        h = jnp.maximum(
            jnp.dot(x, w1, preferred_element_type=jnp.float32) + b1_ref[...],
            0.0,
        ).astype(jnp.bfloat16)
        h = jnp.maximum(
            jnp.dot(h, w2, preferred_element_type=jnp.float32) + b2_ref[...],
            0.0,
        ).astype(jnp.bfloat16)
        ys.append(
            jnp.dot(h, w3, preferred_element_type=jnp.float32) + b3_ref[...]
        )
    y = jnp.stack(ys, axis=1).reshape(_MLP_BLK, _DEEP_OUT)
    o_ref[...] = y.T[None]


def _tc_mlp(emb_pack, W1, b1, W2, b2, W3, b3):
    grid = (_N_FIELDS, _MLP_J)
    return pl.pallas_call(
        _mlp_body,
        grid=grid,
        in_specs=[
            pl.BlockSpec((_MLP_PBLK, 4 * _EMBED_DIM), lambda f, j: (f * _MLP_J + j, 0)),
            pl.BlockSpec((_EMBED_DIM, _H1), lambda f, j: (0, 0)),
            pl.BlockSpec((1, _H1), lambda f, j: (0, 0)),
            pl.BlockSpec((_H1, _H2), lambda f, j: (0, 0)),
            pl.BlockSpec((1, _H2), lambda f, j: (0, 0)),
            pl.BlockSpec((_H2, _DEEP_OUT), lambda f, j: (0, 0)),
            pl.BlockSpec((1, _DEEP_OUT), lambda f, j: (0, 0)),
        ],
        out_specs=pl.BlockSpec((1, _DEEP_OUT, _MLP_BLK), lambda f, j: (f, 0, j)),
        out_shape=jax.ShapeDtypeStruct((_N_FIELDS, _DEEP_OUT, _BATCH), jnp.float32),
    )(
        emb_pack,
        W1,
        b1.reshape(1, _H1),
        W2,
        b2.reshape(1, _H2),
        W3,
        b3.reshape(1, _DEEP_OUT),
    )


def kernel(wide_input, deep_input, table, W_wide, b_wide, W1, b1, W2, b2, W3, b3):
    idx = deep_input.astype(jnp.int32).T.reshape(_B_FLAT)
    table_rm = _tc_table_pack(table.T)
    emb_pack = _sc_gather(table_rm, idx).reshape(_B_PACK, 4 * _EMBED_DIM)
    wide_t = _tc_wide(wide_input, W_wide, b_wide)
    deep_t = _tc_mlp(emb_pack, W1, b1, W2, b2, W3, b3)
    wide_out = wide_t.T
    deep_out = jnp.transpose(deep_t, (2, 0, 1))
    return (wide_out, deep_out)


# speedup vs baseline: 7.3613x; 1.0170x over previous
import functools

import jax
import jax.numpy as jnp
from jax import lax
from jax.experimental import pallas as pl
from jax.experimental.pallas import tpu as pltpu
from jax.experimental.pallas import tpu_sc as plsc

_VOCAB = 1000000
_EMBED_DIM = 32
_BATCH = 4096
_N_FIELDS = 26
_WIDE_IN = 1024
_WIDE_OUT = 64
_H1 = 256
_H2 = 128
_DEEP_OUT = 64

_B_FLAT = _BATCH * _N_FIELDS
_B_PACK = _B_FLAT // 4

_NC = 2
_NS = 16
_NW = _NC * _NS
_B_PER_W = _B_FLAT // _NW


_TR_BLKV = 16384
_TR_STEPS = _VOCAB // _TR_BLKV
_TR_TAIL = _VOCAB - _TR_STEPS * _TR_BLKV


def _pack_body(x_ref, o_ref):
    eye_pad = jnp.concatenate(
        [
            jnp.eye(_EMBED_DIM, dtype=jnp.float32),
            jnp.zeros((_EMBED_DIM, 128 - _EMBED_DIM), dtype=jnp.float32),
        ],
        axis=1,
    )
    o_ref[...] = lax.dot_general(
        x_ref[...], eye_pad, (((0,), (0,)), ((), ())),
        preferred_element_type=jnp.float32,
    )


def _tc_table_pack(table_t):
    grid = (pl.cdiv(_VOCAB, _TR_BLKV),)
    return pl.pallas_call(
        _pack_body,
        grid=grid,
        in_specs=[pl.BlockSpec((_EMBED_DIM, _TR_BLKV), lambda i: (0, i))],
        out_specs=pl.BlockSpec((_TR_BLKV, 128), lambda i: (i, 0)),
        out_shape=jax.ShapeDtypeStruct((_VOCAB, 128), jnp.float32),
    )(table_t)


_G_CHUNKS = 8
_G_ROWS = _B_PER_W // _G_CHUNKS


def _sc_gather(table_rm, idx):
    mesh = plsc.VectorSubcoreMesh(core_axis_name="c", subcore_axis_name="s")

    @functools.partial(
        pl.kernel,
        mesh=mesh,
        out_type=jax.ShapeDtypeStruct((_B_FLAT, _EMBED_DIM), jnp.float32),
        compiler_params=pltpu.CompilerParams(use_tc_tiling_on_sc=False),
        scratch_types=[
            pltpu.VMEM((_B_PER_W,), jnp.int32),
            pltpu.VMEM((_G_ROWS, 128), jnp.float32),
            pltpu.VMEM((_G_ROWS, 128), jnp.float32),
            pltpu.SemaphoreType.DMA,
            pltpu.SemaphoreType.DMA,
        ],
    )
    def k(table_hbm, idx_hbm, out_hbm, idx_v, rows_a, rows_b, sem_a, sem_b):
        wid = lax.axis_index("s") * _NC + lax.axis_index("c")
        base = wid * _B_PER_W
        bufs = (rows_a, rows_b)
        sems = (sem_a, sem_b)
        pltpu.sync_copy(idx_hbm.at[pl.ds(base, _B_PER_W)], idx_v)

        def gather_start(c):
            return pltpu.async_copy(
                table_hbm.at[idx_v.at[pl.ds(c * _G_ROWS, _G_ROWS)]],
                bufs[c % 2],
                sems[c % 2],
            )

        def strip_out(c):
            pltpu.sync_copy(
                bufs[c % 2].at[:, pl.ds(0, _EMBED_DIM)],
                out_hbm.at[pl.ds(base + c * _G_ROWS, _G_ROWS)],
            )

        pending = gather_start(0)
        for c in range(1, _G_CHUNKS):
            pending.wait()
            pending = gather_start(c)
            strip_out(c - 1)
        pending.wait()
        strip_out(_G_CHUNKS - 1)

    return k(table_rm, idx)


_WIDE_BLK = 512


def _wide_body(x_ref, w_ref, b_ref, o_ref):
    y = (
        jnp.dot(x_ref[...], w_ref[...], preferred_element_type=jnp.float32)
        + b_ref[...]
    )
    o_ref[...] = y.T


def _tc_wide(wide_input, W_wide, b_wide):
    grid = (_BATCH // _WIDE_BLK,)
    return pl.pallas_call(
        _wide_body,
        grid=grid,
        in_specs=[
            pl.BlockSpec((_WIDE_BLK, _WIDE_IN), lambda i: (i, 0)),
            pl.BlockSpec((_WIDE_IN, _WIDE_OUT), lambda i: (0, 0)),
            pl.BlockSpec((1, _WIDE_OUT), lambda i: (0, 0)),
        ],
        out_specs=pl.BlockSpec((_WIDE_OUT, _WIDE_BLK), lambda i: (0, i)),
        out_shape=jax.ShapeDtypeStruct((_WIDE_OUT, _BATCH), jnp.float32),
    )(wide_input, W_wide, b_wide.reshape(1, _WIDE_OUT))


_MLP_BLK = 2048
_MLP_PBLK = _MLP_BLK // 4
_MLP_J = _BATCH // _MLP_BLK


def _mlp_body(x_ref, w1_ref, b1_ref, w2_ref, b2_ref, w3_ref, b3_ref, o_ref):
    w1 = w1_ref[...].astype(jnp.bfloat16)
    w2 = w2_ref[...].astype(jnp.bfloat16)
    w3 = w3_ref[...].astype(jnp.bfloat16)
    ys = []
    for k in range(4):
        x = x_ref[:, k * _EMBED_DIM : (k + 1) * _EMBED_DIM].astype(jnp.bfloat16)
        h = jnp.maximum(
            jnp.dot(x, w1, preferred_element_type=jnp.float32) + b1_ref[...],
            0.0,
        ).astype(jnp.bfloat16)
        h = jnp.maximum(
            jnp.dot(h, w2, preferred_element_type=jnp.float32) + b2_ref[...],
            0.0,
        ).astype(jnp.bfloat16)
        ys.append(
            jnp.dot(h, w3, preferred_element_type=jnp.float32) + b3_ref[...]
        )
    y = jnp.stack(ys, axis=1).reshape(_MLP_BLK, _DEEP_OUT)
    o_ref[...] = y.T[None]


def _tc_mlp(emb_pack, W1, b1, W2, b2, W3, b3):
    grid = (_N_FIELDS, _MLP_J)
    return pl.pallas_call(
        _mlp_body,
        grid=grid,
        in_specs=[
            pl.BlockSpec((_MLP_PBLK, 4 * _EMBED_DIM), lambda f, j: (f * _MLP_J + j, 0)),
            pl.BlockSpec((_EMBED_DIM, _H1), lambda f, j: (0, 0)),
            pl.BlockSpec((1, _H1), lambda f, j: (0, 0)),
            pl.BlockSpec((_H1, _H2), lambda f, j: (0, 0)),
            pl.BlockSpec((1, _H2), lambda f, j: (0, 0)),
            pl.BlockSpec((_H2, _DEEP_OUT), lambda f, j: (0, 0)),
            pl.BlockSpec((1, _DEEP_OUT), lambda f, j: (0, 0)),
        ],
        out_specs=pl.BlockSpec((1, _DEEP_OUT, _MLP_BLK), lambda f, j: (f, 0, j)),
        out_shape=jax.ShapeDtypeStruct((_N_FIELDS, _DEEP_OUT, _BATCH), jnp.float32),
    )(
        emb_pack,
        W1,
        b1.reshape(1, _H1),
        W2,
        b2.reshape(1, _H2),
        W3,
        b3.reshape(1, _DEEP_OUT),
    )


def kernel(wide_input, deep_input, table, W_wide, b_wide, W1, b1, W2, b2, W3, b3):
    idx = deep_input.astype(jnp.int32).T.reshape(_B_FLAT)
    table_rm = _tc_table_pack(table.T)
    emb_pack = _sc_gather(table_rm, idx).reshape(_B_PACK, 4 * _EMBED_DIM)
    wide_t = _tc_wide(wide_input, W_wide, b_wide)
    deep_t = _tc_mlp(emb_pack, W1, b1, W2, b2, W3, b3)
    wide_out = wide_t.T
    deep_out = jnp.transpose(deep_t, (2, 0, 1))
    return (wide_out, deep_out)


# speedup vs baseline: 7.9443x; 1.0792x over previous
import functools

import jax
import jax.numpy as jnp
from jax import lax
from jax.experimental import pallas as pl
from jax.experimental.pallas import tpu as pltpu
from jax.experimental.pallas import tpu_sc as plsc

_VOCAB = 1000000
_EMBED_DIM = 32
_BATCH = 4096
_N_FIELDS = 26
_WIDE_IN = 1024
_WIDE_OUT = 64
_H1 = 256
_H2 = 128
_DEEP_OUT = 64

_B_FLAT = _BATCH * _N_FIELDS
_B_PACK = _B_FLAT // 4

_NC = 2
_NS = 16
_NW = _NC * _NS
_B_PER_W = _B_FLAT // _NW


_TR_BLKV = 16384
_TR_STEPS = _VOCAB // _TR_BLKV
_TR_TAIL = _VOCAB - _TR_STEPS * _TR_BLKV


def _pack_body(x_ref, o_ref):
    eye_pad = jnp.concatenate(
        [
            jnp.eye(_EMBED_DIM, dtype=jnp.float32),
            jnp.zeros((_EMBED_DIM, 128 - _EMBED_DIM), dtype=jnp.float32),
        ],
        axis=1,
    )
    o_ref[...] = lax.dot_general(
        x_ref[...], eye_pad, (((0,), (0,)), ((), ())),
        preferred_element_type=jnp.float32,
    )


def _tc_table_pack(table_t):
    grid = (pl.cdiv(_VOCAB, _TR_BLKV),)
    return pl.pallas_call(
        _pack_body,
        grid=grid,
        in_specs=[pl.BlockSpec((_EMBED_DIM, _TR_BLKV), lambda i: (0, i))],
        out_specs=pl.BlockSpec((_TR_BLKV, 128), lambda i: (i, 0)),
        out_shape=jax.ShapeDtypeStruct((_VOCAB, 128), jnp.float32),
    )(table_t)


_G_CHUNKS = 8
_G_ROWS = _B_PER_W // _G_CHUNKS


def _sc_gather(table_rm, idx):
    mesh = plsc.VectorSubcoreMesh(core_axis_name="c", subcore_axis_name="s")

    @functools.partial(
        pl.kernel,
        mesh=mesh,
        out_type=jax.ShapeDtypeStruct((_B_FLAT, _EMBED_DIM), jnp.float32),
        compiler_params=pltpu.CompilerParams(use_tc_tiling_on_sc=False),
        scratch_types=[
            pltpu.VMEM((_B_PER_W,), jnp.int32),
            pltpu.VMEM((_G_ROWS, 128), jnp.float32),
            pltpu.VMEM((_G_ROWS, 128), jnp.float32),
            pltpu.SemaphoreType.DMA,
            pltpu.SemaphoreType.DMA,
        ],
    )
    def k(table_hbm, idx_hbm, out_hbm, idx_v, rows_a, rows_b, sem_a, sem_b):
        wid = lax.axis_index("s") * _NC + lax.axis_index("c")
        base = wid * _B_PER_W
        bufs = (rows_a, rows_b)
        sems = (sem_a, sem_b)
        pltpu.sync_copy(idx_hbm.at[pl.ds(base, _B_PER_W)], idx_v)

        def gather_start(c):
            return pltpu.async_copy(
                table_hbm.at[idx_v.at[pl.ds(c * _G_ROWS, _G_ROWS)]],
                bufs[c % 2],
                sems[c % 2],
            )

        def strip_out(c):
            pltpu.sync_copy(
                bufs[c % 2].at[:, pl.ds(0, _EMBED_DIM)],
                out_hbm.at[pl.ds(base + c * _G_ROWS, _G_ROWS)],
            )

        pending = gather_start(0)
        for c in range(1, _G_CHUNKS):
            pending.wait()
            pending = gather_start(c)
            strip_out(c - 1)
        pending.wait()
        strip_out(_G_CHUNKS - 1)

    return k(table_rm, idx)


_WIDE_BLK = 512


def _wide_body(x_ref, w_ref, b_ref, o_ref):
    y = (
        jnp.dot(x_ref[...], w_ref[...], preferred_element_type=jnp.float32)
        + b_ref[...]
    )
    o_ref[...] = y.T


def _tc_wide(wide_input, W_wide, b_wide):
    grid = (_BATCH // _WIDE_BLK,)
    return pl.pallas_call(
        _wide_body,
        grid=grid,
        in_specs=[
            pl.BlockSpec((_WIDE_BLK, _WIDE_IN), lambda i: (i, 0)),
            pl.BlockSpec((_WIDE_IN, _WIDE_OUT), lambda i: (0, 0)),
            pl.BlockSpec((1, _WIDE_OUT), lambda i: (0, 0)),
        ],
        out_specs=pl.BlockSpec((_WIDE_OUT, _WIDE_BLK), lambda i: (0, i)),
        out_shape=jax.ShapeDtypeStruct((_WIDE_OUT, _BATCH), jnp.float32),
    )(wide_input, W_wide, b_wide.reshape(1, _WIDE_OUT))


_MLP_BLK = 4096
_MLP_PBLK = _MLP_BLK // 4
_MLP_J = _BATCH // _MLP_BLK


def _mlp_body(x_ref, w1_ref, b1_ref, w2_ref, b2_ref, w3_ref, b3_ref, o_ref):
    w1 = w1_ref[...].astype(jnp.bfloat16)
    w2 = w2_ref[...].astype(jnp.bfloat16)
    w3 = w3_ref[...].astype(jnp.bfloat16)
    ys = []
    for k in range(4):
        x = x_ref[:, k * _EMBED_DIM : (k + 1) * _EMBED_DIM].astype(jnp.bfloat16)
        h = jnp.maximum(
            jnp.dot(x, w1, preferred_element_type=jnp.float32) + b1_ref[...],
            0.0,
        ).astype(jnp.bfloat16)
        h = jnp.maximum(
            jnp.dot(h, w2, preferred_element_type=jnp.float32) + b2_ref[...],
            0.0,
        ).astype(jnp.bfloat16)
        ys.append(
            jnp.dot(h, w3, preferred_element_type=jnp.float32) + b3_ref[...]
        )
    y = jnp.stack(ys, axis=1).reshape(_MLP_BLK, _DEEP_OUT)
    o_ref[...] = y.T[None]


def _tc_mlp(emb_pack, W1, b1, W2, b2, W3, b3):
    grid = (_N_FIELDS, _MLP_J)
    return pl.pallas_call(
        _mlp_body,
        grid=grid,
        in_specs=[
            pl.BlockSpec((_MLP_PBLK, 4 * _EMBED_DIM), lambda f, j: (f * _MLP_J + j, 0)),
            pl.BlockSpec((_EMBED_DIM, _H1), lambda f, j: (0, 0)),
            pl.BlockSpec((1, _H1), lambda f, j: (0, 0)),
            pl.BlockSpec((_H1, _H2), lambda f, j: (0, 0)),
            pl.BlockSpec((1, _H2), lambda f, j: (0, 0)),
            pl.BlockSpec((_H2, _DEEP_OUT), lambda f, j: (0, 0)),
            pl.BlockSpec((1, _DEEP_OUT), lambda f, j: (0, 0)),
        ],
        out_specs=pl.BlockSpec((1, _DEEP_OUT, _MLP_BLK), lambda f, j: (f, 0, j)),
        out_shape=jax.ShapeDtypeStruct((_N_FIELDS, _DEEP_OUT, _BATCH), jnp.float32),
    )(
        emb_pack,
        W1,
        b1.reshape(1, _H1),
        W2,
        b2.reshape(1, _H2),
        W3,
        b3.reshape(1, _DEEP_OUT),
    )


def kernel(wide_input, deep_input, table, W_wide, b_wide, W1, b1, W2, b2, W3, b3):
    idx = deep_input.astype(jnp.int32).T.reshape(_B_FLAT)
    table_rm = _tc_table_pack(table.T)
    emb_pack = _sc_gather(table_rm, idx).reshape(_B_PACK, 4 * _EMBED_DIM)
    wide_t = _tc_wide(wide_input, W_wide, b_wide)
    deep_t = _tc_mlp(emb_pack, W1, b1, W2, b2, W3, b3)
    wide_out = wide_t.T
    deep_out = jnp.transpose(deep_t, (2, 0, 1))
    return (wide_out, deep_out)


# speedup vs baseline: 8.0524x; 1.0136x over previous
import functools

import jax
import jax.numpy as jnp
from jax import lax
from jax.experimental import pallas as pl
from jax.experimental.pallas import tpu as pltpu
from jax.experimental.pallas import tpu_sc as plsc

_VOCAB = 1000000
_EMBED_DIM = 32
_BATCH = 4096
_N_FIELDS = 26
_WIDE_IN = 1024
_WIDE_OUT = 64
_H1 = 256
_H2 = 128
_DEEP_OUT = 64

_B_FLAT = _BATCH * _N_FIELDS
_B_PACK = _B_FLAT // 4

_NC = 2
_NS = 16
_NW = _NC * _NS
_B_PER_W = _B_FLAT // _NW


_TR_BLKV = 16384
_TR_STEPS = _VOCAB // _TR_BLKV
_TR_TAIL = _VOCAB - _TR_STEPS * _TR_BLKV


def _pack_body(x_ref, o_ref):
    eye_pad = jnp.concatenate(
        [
            jnp.eye(_EMBED_DIM, dtype=jnp.float32),
            jnp.zeros((_EMBED_DIM, 128 - _EMBED_DIM), dtype=jnp.float32),
        ],
        axis=1,
    )
    o_ref[...] = lax.dot_general(
        x_ref[...], eye_pad, (((0,), (0,)), ((), ())),
        preferred_element_type=jnp.float32,
    )


def _tc_table_pack(table_t):
    grid = (pl.cdiv(_VOCAB, _TR_BLKV),)
    return pl.pallas_call(
        _pack_body,
        grid=grid,
        in_specs=[pl.BlockSpec((_EMBED_DIM, _TR_BLKV), lambda i: (0, i))],
        out_specs=pl.BlockSpec((_TR_BLKV, 128), lambda i: (i, 0)),
        out_shape=jax.ShapeDtypeStruct((_VOCAB, 128), jnp.float32),
    )(table_t)


_G_CHUNKS = 8
_G_ROWS = _B_PER_W // _G_CHUNKS


def _sc_gather(table_rm, idx):
    mesh = plsc.VectorSubcoreMesh(core_axis_name="c", subcore_axis_name="s")

    @functools.partial(
        pl.kernel,
        mesh=mesh,
        out_type=jax.ShapeDtypeStruct((_B_FLAT, _EMBED_DIM), jnp.float32),
        compiler_params=pltpu.CompilerParams(use_tc_tiling_on_sc=False),
        scratch_types=[
            pltpu.VMEM((_B_PER_W,), jnp.int32),
            pltpu.VMEM((_G_ROWS, 128), jnp.float32),
            pltpu.VMEM((_G_ROWS, 128), jnp.float32),
            pltpu.SemaphoreType.DMA,
            pltpu.SemaphoreType.DMA,
        ],
    )
    def k(table_hbm, idx_hbm, out_hbm, idx_v, rows_a, rows_b, sem_a, sem_b):
        wid = lax.axis_index("s") * _NC + lax.axis_index("c")
        base = wid * _B_PER_W
        bufs = (rows_a, rows_b)
        sems = (sem_a, sem_b)
        pltpu.sync_copy(idx_hbm.at[pl.ds(base, _B_PER_W)], idx_v)

        def gather_start(c):
            return pltpu.async_copy(
                table_hbm.at[idx_v.at[pl.ds(c * _G_ROWS, _G_ROWS)]],
                bufs[c % 2],
                sems[c % 2],
            )

        def strip_out(c):
            pltpu.sync_copy(
                bufs[c % 2].at[:, pl.ds(0, _EMBED_DIM)],
                out_hbm.at[pl.ds(base + c * _G_ROWS, _G_ROWS)],
            )

        pending = gather_start(0)
        for c in range(1, _G_CHUNKS):
            pending.wait()
            pending = gather_start(c)
            strip_out(c - 1)
        pending.wait()
        strip_out(_G_CHUNKS - 1)

    return k(table_rm, idx)


_WIDE_BLK = 512


def _wide_body(x_ref, w_ref, b_ref, o_ref):
    y = (
        jnp.dot(x_ref[...], w_ref[...], preferred_element_type=jnp.float32)
        + b_ref[...]
    )
    o_ref[...] = y.T


def _tc_wide(wide_input, W_wide, b_wide):
    grid = (_BATCH // _WIDE_BLK,)
    return pl.pallas_call(
        _wide_body,
        grid=grid,
        in_specs=[
            pl.BlockSpec((_WIDE_BLK, _WIDE_IN), lambda i: (i, 0)),
            pl.BlockSpec((_WIDE_IN, _WIDE_OUT), lambda i: (0, 0)),
            pl.BlockSpec((1, _WIDE_OUT), lambda i: (0, 0)),
        ],
        out_specs=pl.BlockSpec((_WIDE_OUT, _WIDE_BLK), lambda i: (0, i)),
        out_shape=jax.ShapeDtypeStruct((_WIDE_OUT, _BATCH), jnp.float32),
    )(wide_input, W_wide, b_wide.reshape(1, _WIDE_OUT))


_MLP_BLK = 8192
_MLP_PBLK = _MLP_BLK // 4


def _mlp_body(x_ref, w1_ref, b1_ref, w2_ref, b2_ref, w3_ref, b3_ref, o_ref):
    w1 = w1_ref[...].astype(jnp.bfloat16)
    w2 = w2_ref[...].astype(jnp.bfloat16)
    w3 = w3_ref[...].astype(jnp.bfloat16)
    ys = []
    for k in range(4):
        x = x_ref[:, k * _EMBED_DIM : (k + 1) * _EMBED_DIM].astype(jnp.bfloat16)
        h = jnp.maximum(
            jnp.dot(x, w1, preferred_element_type=jnp.float32) + b1_ref[...],
            0.0,
        ).astype(jnp.bfloat16)
        h = jnp.maximum(
            jnp.dot(h, w2, preferred_element_type=jnp.float32) + b2_ref[...],
            0.0,
        ).astype(jnp.bfloat16)
        ys.append(
            jnp.dot(h, w3, preferred_element_type=jnp.float32) + b3_ref[...]
        )
    y = jnp.stack(ys, axis=1).reshape(_MLP_BLK, _DEEP_OUT)
    o_ref[...] = jnp.stack(
        [y[: _MLP_BLK // 2].T, y[_MLP_BLK // 2 :].T], axis=0
    )


def _tc_mlp(emb_pack, W1, b1, W2, b2, W3, b3):
    grid = (_N_FIELDS // 2,)
    return pl.pallas_call(
        _mlp_body,
        grid=grid,
        in_specs=[
            pl.BlockSpec((_MLP_PBLK, 4 * _EMBED_DIM), lambda f: (f, 0)),
            pl.BlockSpec((_EMBED_DIM, _H1), lambda f: (0, 0)),
            pl.BlockSpec((1, _H1), lambda f: (0, 0)),
            pl.BlockSpec((_H1, _H2), lambda f: (0, 0)),
            pl.BlockSpec((1, _H2), lambda f: (0, 0)),
            pl.BlockSpec((_H2, _DEEP_OUT), lambda f: (0, 0)),
            pl.BlockSpec((1, _DEEP_OUT), lambda f: (0, 0)),
        ],
        out_specs=pl.BlockSpec((2, _DEEP_OUT, _BATCH), lambda f: (f, 0, 0)),
        out_shape=jax.ShapeDtypeStruct((_N_FIELDS, _DEEP_OUT, _BATCH), jnp.float32),
    )(
        emb_pack,
        W1,
        b1.reshape(1, _H1),
        W2,
        b2.reshape(1, _H2),
        W3,
        b3.reshape(1, _DEEP_OUT),
    )


def kernel(wide_input, deep_input, table, W_wide, b_wide, W1, b1, W2, b2, W3, b3):
    idx = deep_input.astype(jnp.int32).T.reshape(_B_FLAT)
    table_rm = _tc_table_pack(table.T)
    emb_pack = _sc_gather(table_rm, idx).reshape(_B_PACK, 4 * _EMBED_DIM)
    wide_t = _tc_wide(wide_input, W_wide, b_wide)
    deep_t = _tc_mlp(emb_pack, W1, b1, W2, b2, W3, b3)
    wide_out = wide_t.T
    deep_out = jnp.transpose(deep_t, (2, 0, 1))
    return (wide_out, deep_out)
